# Initial kernel scaffold; baseline (speedup 1.0000x reference)
import functools

import jax
import jax.numpy as jnp
from jax import lax
from jax.experimental import pallas as pl
from jax.experimental.pallas import tpu as pltpu
from jax.experimental.pallas import tpu_sc as plsc

N_ROWS = 4096 * 50
D_EMB = 128
D_HID = 1024
N_WORKERS = 32
CHUNK = 128
N_CHUNKS = N_ROWS // (N_WORKERS * CHUNK)

_sc_mesh = plsc.VectorSubcoreMesh(core_axis_name="c", subcore_axis_name="s")


@functools.partial(
    pl.kernel,
    out_type=jax.ShapeDtypeStruct((N_ROWS, D_EMB), jnp.float32),
    mesh=_sc_mesh,
    scratch_types=[
        pltpu.VMEM((N_CHUNKS, CHUNK), jnp.int32),
        pltpu.VMEM((2, CHUNK, D_EMB), jnp.float32),
        pltpu.SemaphoreType.DMA,
        pltpu.SemaphoreType.DMA,
    ],
)
def _sc_gather(ids_hbm, table_hbm, out_hbm, idx_v, rows_v, gsem, ssem):
    wid = lax.axis_index("s") * 2 + lax.axis_index("c")
    pltpu.sync_copy(ids_hbm.at[pl.ds(wid * N_CHUNKS, N_CHUNKS)], idx_v)
    base_row = wid * N_CHUNKS * CHUNK

    def chunk_pair(j2, carry):
        for b in range(2):
            j = j2 + b

            @pl.when(j < N_CHUNKS)
            def _():
                pltpu.async_copy(
                    table_hbm.at[idx_v.at[j]], rows_v.at[b], gsem
                ).wait()
                pltpu.sync_copy(
                    rows_v.at[b],
                    out_hbm.at[pl.ds(base_row + j * CHUNK, CHUNK)],
                )
        return carry

    lax.fori_loop(0, (N_CHUNKS + 1) // 2, lambda i, c: chunk_pair(i * 2, c), 0)


ROW_BLK = 512


def _proj_ln_body(emb_ref, w_ref, g_ref, b_ref, out_ref):
    x = emb_ref[...]
    p = jnp.dot(x, w_ref[...], preferred_element_type=jnp.float32)
    mu = jnp.mean(p, axis=-1, keepdims=True)
    var = jnp.mean((p - mu) ** 2, axis=-1, keepdims=True)
    inv = lax.rsqrt(var + 1e-12)
    out_ref[...] = (p - mu) * inv * g_ref[...] + b_ref[...]


_proj_ln = pl.pallas_call(
    _proj_ln_body,
    grid=(N_ROWS // ROW_BLK,),
    in_specs=[
        pl.BlockSpec((ROW_BLK, D_EMB), lambda i: (i, 0)),
        pl.BlockSpec((D_EMB, D_HID), lambda i: (0, 0)),
        pl.BlockSpec((1, D_HID), lambda i: (0, 0)),
        pl.BlockSpec((1, D_HID), lambda i: (0, 0)),
    ],
    out_specs=pl.BlockSpec((ROW_BLK, D_HID), lambda i: (i, 0)),
    out_shape=jax.ShapeDtypeStruct((N_ROWS, D_HID), jnp.float32),
)


def kernel(entity_ids, table, W, gamma, beta):
    ids2d = entity_ids.reshape(N_WORKERS * N_CHUNKS, CHUNK)
    rows = _sc_gather(ids2d, table)
    out = _proj_ln(rows, W, gamma.reshape(1, D_HID), beta.reshape(1, D_HID))
    return out.reshape(4096, 50, D_HID)


# speedup vs baseline: 1.3257x; 1.3257x over previous
import functools

import jax
import jax.numpy as jnp
from jax import lax
from jax.experimental import pallas as pl
from jax.experimental.pallas import tpu as pltpu
from jax.experimental.pallas import tpu_sc as plsc

N_ROWS = 4096 * 50
D_EMB = 128
D_HID = 1024
N_WORKERS = 32
CHUNK = 128
N_CHUNKS = N_ROWS // (N_WORKERS * CHUNK)

@functools.cache
def _make_sc_gather():
    mesh = plsc.VectorSubcoreMesh(core_axis_name="c", subcore_axis_name="s")

    @functools.partial(
        pl.kernel,
        out_type=jax.ShapeDtypeStruct((N_ROWS, D_EMB), jnp.float32),
        mesh=mesh,
        scratch_types=[
            pltpu.VMEM((N_CHUNKS * CHUNK,), jnp.int32),
            pltpu.VMEM((2, CHUNK, D_EMB), jnp.float32),
            pltpu.SemaphoreType.DMA,
            pltpu.SemaphoreType.DMA,
        ],
    )
    def _sc_gather(ids_hbm, table_hbm, out_hbm, idx_v, rows_v, gsem, ssem):
        wid = lax.axis_index("s") * 2 + lax.axis_index("c")
        pltpu.sync_copy(ids_hbm.at[pl.ds(wid * N_CHUNKS * CHUNK, N_CHUNKS * CHUNK)], idx_v)
        base_row = wid * N_CHUNKS * CHUNK

        def chunk_pair(j2, carry):
            for b in range(2):
                j = j2 + b

                @pl.when(j < N_CHUNKS)
                def _():
                    pltpu.async_copy(
                        table_hbm.at[idx_v.at[pl.ds(j * CHUNK, CHUNK)]],
                        rows_v.at[b],
                        gsem,
                    ).wait()
                    pltpu.sync_copy(
                        rows_v.at[b],
                        out_hbm.at[pl.ds(base_row + j * CHUNK, CHUNK)],
                    )
            return carry

        lax.fori_loop(
            0, (N_CHUNKS + 1) // 2, lambda i, c: chunk_pair(i * 2, c), 0
        )

    return _sc_gather


ROW_BLK = 512


def _proj_ln_body(emb_ref, w_ref, g_ref, b_ref, out_ref):
    x = emb_ref[...]
    p = jnp.dot(x, w_ref[...], preferred_element_type=jnp.float32)
    mu = jnp.mean(p, axis=-1, keepdims=True)
    var = jnp.mean((p - mu) ** 2, axis=-1, keepdims=True)
    inv = lax.rsqrt(var + 1e-12)
    out_ref[...] = (p - mu) * inv * g_ref[...] + b_ref[...]


_proj_ln = pl.pallas_call(
    _proj_ln_body,
    grid=(N_ROWS // ROW_BLK,),
    in_specs=[
        pl.BlockSpec((ROW_BLK, D_EMB), lambda i: (i, 0)),
        pl.BlockSpec((D_EMB, D_HID), lambda i: (0, 0)),
        pl.BlockSpec((1, D_HID), lambda i: (0, 0)),
        pl.BlockSpec((1, D_HID), lambda i: (0, 0)),
    ],
    out_specs=pl.BlockSpec((ROW_BLK, D_HID), lambda i: (i, 0)),
    out_shape=jax.ShapeDtypeStruct((N_ROWS, D_HID), jnp.float32),
)


def kernel(entity_ids, table, W, gamma, beta):
    ids_flat = entity_ids.reshape(N_ROWS)
    rows = _make_sc_gather()(ids_flat, table)
    out = _proj_ln(rows, W, gamma.reshape(1, D_HID), beta.reshape(1, D_HID))
    return out.reshape(4096, 50, D_HID)


# speedup vs baseline: 1.7100x; 1.2899x over previous
import functools

import jax
import jax.numpy as jnp
from jax import lax
from jax.experimental import pallas as pl
from jax.experimental.pallas import tpu as pltpu
from jax.experimental.pallas import tpu_sc as plsc

N_ROWS = 4096 * 50
D_EMB = 128
D_HID = 1024
N_WORKERS = 32
CHUNK = 128
N_CHUNKS = N_ROWS // (N_WORKERS * CHUNK)

@functools.cache
def _make_sc_gather():
    mesh = plsc.VectorSubcoreMesh(core_axis_name="c", subcore_axis_name="s")

    @functools.partial(
        pl.kernel,
        out_type=jax.ShapeDtypeStruct((N_ROWS, D_EMB), jnp.float32),
        mesh=mesh,
        scratch_types=[
            pltpu.VMEM((N_CHUNKS * CHUNK,), jnp.int32),
            pltpu.VMEM((2, CHUNK, D_EMB), jnp.float32),
            pltpu.SemaphoreType.DMA,
            pltpu.SemaphoreType.DMA,
        ],
    )
    def _sc_gather(ids_hbm, table_hbm, out_hbm, idx_v, rows_v, gsem, ssem):
        wid = lax.axis_index("s") * 2 + lax.axis_index("c")
        pltpu.sync_copy(ids_hbm.at[pl.ds(wid * N_CHUNKS * CHUNK, N_CHUNKS * CHUNK)], idx_v)
        base_row = wid * N_CHUNKS * CHUNK

        def chunk_pair(j2, carry):
            for b in range(2):
                j = j2 + b

                @pl.when(j < N_CHUNKS)
                def _():
                    pltpu.async_copy(
                        table_hbm.at[idx_v.at[pl.ds(j * CHUNK, CHUNK)]],
                        rows_v.at[b],
                        gsem,
                    ).wait()
                    pltpu.sync_copy(
                        rows_v.at[b],
                        out_hbm.at[pl.ds(base_row + j * CHUNK, CHUNK)],
                    )
            return carry

        lax.fori_loop(
            0, (N_CHUNKS + 1) // 2, lambda i, c: chunk_pair(i * 2, c), 0
        )

    return _sc_gather


SENT_BLK = 8
SEQ = 50
ROW_BLK = SENT_BLK * SEQ


def _proj_ln_body(emb_ref, w_ref, g_ref, b_ref, out_ref):
    x = emb_ref[...]
    p = jnp.dot(x, w_ref[...], preferred_element_type=jnp.float32)
    mu = jnp.mean(p, axis=-1, keepdims=True)
    var = jnp.mean((p - mu) ** 2, axis=-1, keepdims=True)
    inv = lax.rsqrt(var + 1e-12)
    y = (p - mu) * inv * g_ref[...] + b_ref[...]
    out_ref[...] = y.reshape(SENT_BLK, SEQ, D_HID)


_proj_ln = pl.pallas_call(
    _proj_ln_body,
    grid=(N_ROWS // ROW_BLK,),
    in_specs=[
        pl.BlockSpec((ROW_BLK, D_EMB), lambda i: (i, 0)),
        pl.BlockSpec((D_EMB, D_HID), lambda i: (0, 0)),
        pl.BlockSpec((1, D_HID), lambda i: (0, 0)),
        pl.BlockSpec((1, D_HID), lambda i: (0, 0)),
    ],
    out_specs=pl.BlockSpec((SENT_BLK, SEQ, D_HID), lambda i: (i, 0, 0)),
    out_shape=jax.ShapeDtypeStruct((4096, SEQ, D_HID), jnp.float32),
)


def kernel(entity_ids, table, W, gamma, beta):
    ids_flat = entity_ids.reshape(N_ROWS)
    rows = _make_sc_gather()(ids_flat, table)
    return _proj_ln(rows, W, gamma.reshape(1, D_HID), beta.reshape(1, D_HID))
